# baseline (speedup 1.0000x reference)
import functools

import jax
import jax.numpy as jnp
from jax import lax
from jax.experimental import pallas as pl
from jax.experimental.pallas import tpu as pltpu
from jax.experimental.pallas import tpu_sc as plsc

_CHUNK = 128
_LANES = 16


def _make_gather(B, D, NC, NS):
    NW = NC * NS
    b_per_w = B // NW
    n_chunks = b_per_w // _CHUNK
    pack = 128 // D
    pack_shift = pack.bit_length() - 1
    d_shift = D.bit_length() - 1
    mesh = plsc.VectorSubcoreMesh(core_axis_name="c", subcore_axis_name="s")

    @functools.partial(
        pl.kernel,
        mesh=mesh,
        compiler_params=pltpu.CompilerParams(needs_layout_passes=False),
        out_type=[
            jax.ShapeDtypeStruct((B * D,), jnp.float32),
            jax.ShapeDtypeStruct((B * D,), jnp.float32),
        ],
        scratch_types=[
            pltpu.VMEM((b_per_w,), jnp.int32),
            pltpu.VMEM((b_per_w,), jnp.int32),
            pltpu.VMEM((b_per_w, 128), jnp.float32),
            pltpu.VMEM((b_per_w * D,), jnp.float32),
            pltpu.VMEM((b_per_w * D,), jnp.float32),
            pltpu.SemaphoreType.DMA,
        ],
    )
    def gather2(idx_hbm, ktab_hbm, vtab_hbm, kout_hbm, vout_hbm,
                idx_v, hi_v, buf, kout_v, vout_v, sem):
        wid = lax.axis_index("s") * NC + lax.axis_index("c")
        base = wid * b_per_w
        pltpu.sync_copy(idx_hbm.at[pl.ds(base, b_per_w)], idx_v)

        def hi_body(t, carry):
            sl = pl.ds(t * _LANES, _LANES)
            hi_v[sl] = lax.shift_right_logical(idx_v[sl], pack_shift)
            return carry

        lax.fori_loop(0, b_per_w // _LANES, hi_body, 0, unroll=4)

        lanes = lax.iota(jnp.int32, _LANES)
        for tab_hbm, out_v, out_hbm in (
            (ktab_hbm, kout_v, kout_hbm),
            (vtab_hbm, vout_v, vout_hbm),
        ):
            copies = []
            for c in range(n_chunks):
                sl = pl.ds(c * _CHUNK, _CHUNK)
                copies.append(
                    pltpu.async_copy(tab_hbm.at[hi_v.at[sl]], buf.at[sl], sem))
            for cp in copies:
                cp.wait()

            def ex_body(t, carry):
                j0 = t * _LANES
                idxv = idx_v[pl.ds(j0, _LANES)]
                qb = lax.shift_left(idxv & (pack - 1), d_shift)
                jv = j0 + lanes
                dstb = lax.shift_left(jv, d_shift)
                for c in range(D):
                    val = plsc.load_gather(buf, [jv, qb + c])
                    plsc.store_scatter(out_v, [dstb + c], val)
                return carry

            lax.fori_loop(0, b_per_w // _LANES, ex_body, 0)
            pltpu.sync_copy(out_v, out_hbm.at[pl.ds(base * D, b_per_w * D)])

    return gather2


def kernel(x, key_table, value_table):
    B = x.shape[0]
    N, D = key_table.shape
    info = plsc.get_sparse_core_info()
    fn = _make_gather(B, D, info.num_cores, info.num_subcores)
    pack = 128 // D
    xi = x.astype(jnp.int32).reshape(-1)
    ktab4 = key_table.reshape(N // pack, 128)
    vtab4 = value_table.reshape(N // pack, 128)
    kout, vout = fn(xi, ktab4, vtab4)
    return (kout.reshape(B, D), vout.reshape(B, D))


# speedup vs baseline: 3.7451x; 3.7451x over previous
import functools

import jax
import jax.numpy as jnp
from jax import lax
from jax.experimental import pallas as pl
from jax.experimental.pallas import tpu as pltpu
from jax.experimental.pallas import tpu_sc as plsc

_RING = 8
_G = 16


def _lane(vec, l):
    return lax.squeeze(lax.slice(vec, (l,), (l + 1,)), (0,))


def _make_gather(B, D, N, NC, NS):
    NW = NC * NS
    b_per_w = B // NW
    n_groups = b_per_w // _G
    tail = N % 128
    tail_start = N - tail
    mesh = plsc.VectorSubcoreMesh(core_axis_name="c", subcore_axis_name="s")

    @functools.partial(
        pl.kernel,
        mesh=mesh,
        compiler_params=pltpu.CompilerParams(needs_layout_passes=False),
        out_type=[
            jax.ShapeDtypeStruct((B * D,), jnp.float32),
            jax.ShapeDtypeStruct((B * D,), jnp.float32),
        ],
        scratch_types=[
            pltpu.VMEM((b_per_w,), jnp.int32),
            pltpu.VMEM((_RING, D, 128), jnp.float32),
            pltpu.VMEM((_RING, D, 128), jnp.float32),
            pltpu.VMEM((tail * D,), jnp.float32),
            pltpu.VMEM((tail * D,), jnp.float32),
            pltpu.VMEM((b_per_w * D,), jnp.float32),
            pltpu.VMEM((b_per_w * D,), jnp.float32),
            pltpu.SemaphoreType.DMA((_RING,)),
            pltpu.SemaphoreType.DMA((_RING,)),
        ],
    )
    def gather2(idx_hbm, ktT_hbm, vtT_hbm, ktail_hbm, vtail_hbm,
                kout_hbm, vout_hbm,
                idx_v, bufk, bufv, tailk_v, tailv_v,
                outk_v, outv_v, ksem, vsem):
        wid = lax.axis_index("s") * NC + lax.axis_index("c")
        base = wid * b_per_w
        pltpu.sync_copy(idx_hbm.at[pl.ds(base, b_per_w)], idx_v)
        pltpu.sync_copy(ktail_hbm, tailk_v)
        pltpu.sync_copy(vtail_hbm, tailv_v)

        c0 = lax.iota(jnp.int32, 16)
        c1 = c0 + 16

        def fire(i, slot):
            blk = jnp.where(i < tail_start, lax.shift_right_logical(i, 7), 0)
            off = pl.multiple_of(blk * 128, 128)
            pltpu.async_copy(
                ktT_hbm.at[:, pl.ds(off, 128)], bufk.at[slot], ksem.at[slot])
            pltpu.async_copy(
                vtT_hbm.at[:, pl.ds(off, 128)], bufv.at[slot], vsem.at[slot])

        vec0 = idx_v[pl.ds(0, _G)]
        for l in range(_RING):
            fire(_lane(vec0, l), l)

        def group(g, carry):
            vec = idx_v[pl.ds(g * _G, _G)]
            nstart = jnp.where(g < n_groups - 1, (g + 1) * _G, 0)
            nvec = idx_v[pl.ds(nstart, _G)]
            for l in range(_G):
                slot = l % _RING
                pltpu.make_async_copy(
                    ktT_hbm.at[:, pl.ds(0, 128)], bufk.at[slot], ksem.at[slot]
                ).wait()
                pltpu.make_async_copy(
                    vtT_hbm.at[:, pl.ds(0, 128)], bufv.at[slot], vsem.at[slot]
                ).wait()
                i = _lane(vec, l)
                slot_v = jnp.full((16,), slot, jnp.int32)
                lo_v = jnp.full((16,), i & 127, jnp.int32)
                is_tail = jnp.full((16,), i >= tail_start, jnp.bool_)
                toff = jnp.maximum(i - tail_start, 0) * D
                ob = g * (_G * D) + l * D
                for buf, tail_v, out_v in (
                    (bufk, tailk_v, outk_v),
                    (bufv, tailv_v, outv_v),
                ):
                    v0 = plsc.load_gather(buf, [slot_v, c0, lo_v])
                    v1 = plsc.load_gather(buf, [slot_v, c1, lo_v])
                    t0 = plsc.load_gather(tail_v, [toff + c0])
                    t1 = plsc.load_gather(tail_v, [toff + c1])
                    out_v[pl.ds(ob, 16)] = jnp.where(is_tail, t0, v0)
                    out_v[pl.ds(ob + 16, 16)] = jnp.where(is_tail, t1, v1)
                if l < _RING:
                    fire(_lane(vec, l + _RING), slot)
                else:

                    @pl.when(g < n_groups - 1)
                    def _():
                        fire(_lane(nvec, l - _RING), slot)

            return carry

        lax.fori_loop(0, n_groups, group, 0)
        pltpu.sync_copy(outk_v, kout_hbm.at[pl.ds(base * D, b_per_w * D)])
        pltpu.sync_copy(outv_v, vout_hbm.at[pl.ds(base * D, b_per_w * D)])

    return gather2


def kernel(x, key_table, value_table):
    B = x.shape[0]
    N, D = key_table.shape
    info = plsc.get_sparse_core_info()
    fn = _make_gather(B, D, N, info.num_cores, info.num_subcores)
    tail_start = N - (N % 128)
    xi = x.astype(jnp.int32).reshape(-1)
    ktail = key_table[tail_start:].reshape(-1)
    vtail = value_table[tail_start:].reshape(-1)
    kout, vout = fn(xi, key_table.T, value_table.T, ktail, vtail)
    return (kout.reshape(B, D), vout.reshape(B, D))


# speedup vs baseline: 5.1339x; 1.3708x over previous
import functools

import jax
import jax.numpy as jnp
from jax import lax
from jax.experimental import pallas as pl
from jax.experimental.pallas import tpu as pltpu
from jax.experimental.pallas import tpu_sc as plsc

_RING = 8
_ORING = 16


def _lane(vec, l):
    return lax.squeeze(lax.slice(vec, (l,), (l + 1,)), (0,))


def _make_gather(B, D, N, NC, NS):
    NW = NC * NS
    n_full = N // 128
    n_blocks = n_full + (1 if N % 128 else 0)
    tail = N % 128
    tail_start = N - tail
    nullh = B
    q, r = divmod(n_blocks, NW)
    mesh = plsc.VectorSubcoreMesh(core_axis_name="c", subcore_axis_name="s")

    @functools.partial(
        pl.kernel,
        mesh=mesh,
        compiler_params=pltpu.CompilerParams(needs_layout_passes=False),
        out_type=[
            jax.ShapeDtypeStruct((B * D,), jnp.float32),
            jax.ShapeDtypeStruct((B * D,), jnp.float32),
        ],
        scratch_types=[
            pltpu.VMEM((B,), jnp.int32),
            pltpu.VMEM((B + 16,), jnp.int32),
            pltpu.VMEM((B + 16,), jnp.int32),
            pltpu.VMEM((256,), jnp.int32),
            pltpu.VMEM((16,), jnp.int32),
            pltpu.VMEM((_RING, D, 128), jnp.float32),
            pltpu.VMEM((_RING, D, 128), jnp.float32),
            pltpu.VMEM((tail * D,), jnp.float32),
            pltpu.VMEM((tail * D,), jnp.float32),
            pltpu.VMEM((_ORING * D,), jnp.float32),
            pltpu.VMEM((_ORING * D,), jnp.float32),
            pltpu.SemaphoreType.DMA((_RING,)),
            pltpu.SemaphoreType.DMA((_RING,)),
            pltpu.SemaphoreType.DMA,
            pltpu.SemaphoreType.DMA,
        ],
    )
    def gather2(idx_hbm, ktT_hbm, vtT_hbm, ktail_hbm, vtail_hbm,
                kout_hbm, vout_hbm,
                idx_all, jn, il, head, hs_v, bufk, bufv,
                tailk_v, tailv_v, stgk, stgv,
                ksem, vsem, osemk, osemv):
        wid = lax.axis_index("s") * NC + lax.axis_index("c")
        lo = wid * q + jnp.minimum(wid, r)
        nblk = q + jnp.where(wid < r, 1, 0)
        hi = lo + nblk
        pltpu.sync_copy(idx_hbm, idx_all)
        pltpu.sync_copy(ktail_hbm, tailk_v)
        pltpu.sync_copy(vtail_hbm, tailv_v)

        c0 = lax.iota(jnp.int32, 16)
        c1 = c0 + 16
        lane0 = c0 == 0
        zeros = jnp.zeros((16,), jnp.int32)
        for s in range(16):
            head[pl.ds(s * 16, 16)] = zeros + nullh
        hs_v[pl.ds(0, 16)] = zeros

        def collect(g, cnt):
            vec = idx_all[pl.ds(g * 16, 16)]
            blk = lax.shift_right_logical(vec, 7)
            m = (blk >= lo) & (blk < hi)
            cum = plsc.cumsum(m.astype(jnp.int32))
            pos = cnt + cum - 1
            plsc.store_scatter(jn, [pos], g * 16 + c0, mask=m)
            plsc.store_scatter(il, [pos], vec, mask=m)
            return cnt + _lane(cum, 15)

        cnt = lax.fori_loop(0, B // 16, collect, 0)

        def link(h, carry):
            nv = _lane(jn[pl.ds(h, 16)], 0)
            i = _lane(il[pl.ds(h, 16)], 0)
            b = lax.shift_right_logical(i, 7) - lo
            b_v = jnp.full((16,), b, jnp.int32)
            old = _lane(plsc.load_gather(head, [b_v]), 0)
            packed = nv | lax.shift_left(old, 14)
            plsc.store_scatter(jn, [jnp.full((16,), h, jnp.int32)],
                               jnp.full((16,), packed, jnp.int32), mask=lane0)
            plsc.store_scatter(head, [b_v],
                               jnp.full((16,), h, jnp.int32), mask=lane0)
            return carry

        lax.fori_loop(0, cnt, link, 0)

        def fire(bi, slot):
            blk_eff = jnp.minimum(lo + bi, n_full - 1)
            off = pl.multiple_of(blk_eff * 128, 128)
            pltpu.async_copy(
                ktT_hbm.at[:, pl.ds(off, 128)], bufk.at[slot], ksem.at[slot])
            pltpu.async_copy(
                vtT_hbm.at[:, pl.ds(off, 128)], bufv.at[slot], vsem.at[slot])

        for s in range(_RING):
            fire(s, s)

        def visit(h):
            nv = _lane(jn[pl.ds(h, 16)], 0)
            j = nv & (nullh - 1)
            nxt = lax.shift_right_logical(nv, 14)
            i = _lane(il[pl.ds(h, 16)], 0)
            return j, nxt, i

        def block_body(v, carry):
            for l in range(16):
                bi = v * 16 + l
                slot = l % _RING

                @pl.when(bi < nblk)
                def _():
                    pltpu.make_async_copy(
                        ktT_hbm.at[:, pl.ds(0, 128)], bufk.at[slot],
                        ksem.at[slot]).wait()
                    pltpu.make_async_copy(
                        vtT_hbm.at[:, pl.ds(0, 128)], bufv.at[slot],
                        vsem.at[slot]).wait()
                    bi_v = jnp.full((16,), bi, jnp.int32)
                    h0 = _lane(plsc.load_gather(head, [bi_v]), 0)
                    slot_v = jnp.full((16,), slot, jnp.int32)

                    def walk_body(carry):
                        h, steps = carry
                        j, nxt, i = visit(h)
                        lo_v = jnp.full((16,), i & 127, jnp.int32)
                        is_tail = jnp.full((16,), i >= tail_start, jnp.bool_)
                        toff = jnp.maximum(i - tail_start, 0) * D
                        hs = _lane(hs_v[pl.ds(0, 16)], 0)
                        so = hs & (_ORING - 1)

                        @pl.when((so == 0) & (hs >= _ORING))
                        def _():
                            for _s in range(_ORING):
                                pltpu.make_async_copy(
                                    stgk.at[pl.ds(0, D)],
                                    kout_hbm.at[pl.ds(0, D)], osemk).wait()
                                pltpu.make_async_copy(
                                    stgv.at[pl.ds(0, D)],
                                    vout_hbm.at[pl.ds(0, D)], osemv).wait()

                        for buf, tail_v, stg, out_hbm, osem in (
                            (bufk, tailk_v, stgk, kout_hbm, osemk),
                            (bufv, tailv_v, stgv, vout_hbm, osemv),
                        ):
                            v0 = plsc.load_gather(buf, [slot_v, c0, lo_v])
                            v1 = plsc.load_gather(buf, [slot_v, c1, lo_v])
                            t0 = plsc.load_gather(tail_v, [toff + c0])
                            t1 = plsc.load_gather(tail_v, [toff + c1])
                            stg[pl.ds(so * D, 16)] = jnp.where(is_tail, t0, v0)
                            stg[pl.ds(so * D + 16, 16)] = jnp.where(
                                is_tail, t1, v1)
                            pltpu.async_copy(
                                stg.at[pl.ds(so * D, D)],
                                out_hbm.at[pl.ds(j * D, D)], osem)
                        plsc.store_scatter(
                            hs_v, [zeros],
                            jnp.full((16,), hs + 1, jnp.int32), mask=lane0)
                        return nxt, steps + 1

                    lax.while_loop(
                        lambda c: (c[0] != nullh) & (c[1] < B),
                        walk_body, (h0, 0))

                    @pl.when(bi + _RING < nblk)
                    def _():
                        fire(bi + _RING, slot)

            return carry

        lax.fori_loop(0, 16, block_body, 0)

        hs_end = _lane(hs_v[pl.ds(0, 16)], 0)
        r_end = jnp.where(
            hs_end > 0, ((hs_end - 1) & (_ORING - 1)) + 1, 0)

        def final_drain(_, carry):
            pltpu.make_async_copy(
                stgk.at[pl.ds(0, D)], kout_hbm.at[pl.ds(0, D)], osemk).wait()
            pltpu.make_async_copy(
                stgv.at[pl.ds(0, D)], vout_hbm.at[pl.ds(0, D)], osemv).wait()
            return carry

        lax.fori_loop(0, r_end, final_drain, 0)

    return gather2


def kernel(x, key_table, value_table):
    B = x.shape[0]
    N, D = key_table.shape
    info = plsc.get_sparse_core_info()
    fn = _make_gather(B, D, N, info.num_cores, info.num_subcores)
    tail_start = N - (N % 128)
    xi = x.astype(jnp.int32).reshape(-1)
    ktail = key_table[tail_start:].reshape(-1)
    vtail = value_table[tail_start:].reshape(-1)
    kout, vout = fn(xi, key_table.T, value_table.T, ktail, vtail)
    return (kout.reshape(B, D), vout.reshape(B, D))


# speedup vs baseline: 5.1763x; 1.0083x over previous
import functools

import jax
import jax.numpy as jnp
from jax import lax
from jax.experimental import pallas as pl
from jax.experimental.pallas import tpu as pltpu
from jax.experimental.pallas import tpu_sc as plsc

_RING = 8
_ORING = 16


def _lane(vec, l):
    return lax.squeeze(lax.slice(vec, (l,), (l + 1,)), (0,))


def _make_gather(B, D, N, NC, NS):
    NW = NC * NS
    n_full = N // 128
    n_blocks = n_full + (1 if N % 128 else 0)
    tail = N % 128
    tail_start = N - tail
    nullh = B
    q, r = divmod(n_blocks, NW)
    mesh = plsc.VectorSubcoreMesh(core_axis_name="c", subcore_axis_name="s")

    @functools.partial(
        pl.kernel,
        mesh=mesh,
        compiler_params=pltpu.CompilerParams(needs_layout_passes=False),
        out_type=[
            jax.ShapeDtypeStruct((B * D,), jnp.float32),
            jax.ShapeDtypeStruct((B * D,), jnp.float32),
        ],
        scratch_types=[
            pltpu.VMEM((B,), jnp.int32),
            pltpu.VMEM((B + 16,), jnp.int32),
            pltpu.VMEM((B + 16,), jnp.int32),
            pltpu.VMEM((256,), jnp.int32),
            pltpu.VMEM((16,), jnp.int32),
            pltpu.VMEM((_RING, D, 128), jnp.float32),
            pltpu.VMEM((_RING, D, 128), jnp.float32),
            pltpu.VMEM((tail * D,), jnp.float32),
            pltpu.VMEM((tail * D,), jnp.float32),
            pltpu.VMEM((_ORING * D,), jnp.float32),
            pltpu.VMEM((_ORING * D,), jnp.float32),
            pltpu.SemaphoreType.DMA((_RING,)),
            pltpu.SemaphoreType.DMA((_RING,)),
            pltpu.SemaphoreType.DMA,
            pltpu.SemaphoreType.DMA,
        ],
    )
    def gather2(idx_hbm, ktT_hbm, vtT_hbm, ktail_hbm, vtail_hbm,
                kout_hbm, vout_hbm,
                idx_all, jn, il, head, hs_v, bufk, bufv,
                tailk_v, tailv_v, stgk, stgv,
                ksem, vsem, osemk, osemv):
        wid = lax.axis_index("s") * NC + lax.axis_index("c")
        lo = wid * q + jnp.minimum(wid, r)
        nblk = q + jnp.where(wid < r, 1, 0)
        hi = lo + nblk
        pltpu.sync_copy(idx_hbm, idx_all)
        pltpu.sync_copy(ktail_hbm, tailk_v)
        pltpu.sync_copy(vtail_hbm, tailv_v)

        c0 = lax.iota(jnp.int32, 16)
        c1 = c0 + 16
        lane0 = c0 == 0
        zeros = jnp.zeros((16,), jnp.int32)
        for s in range(16):
            head[pl.ds(s * 16, 16)] = zeros + nullh
        hs_v[pl.ds(0, 16)] = zeros

        def collect(g, cnt):
            vec = idx_all[pl.ds(g * 16, 16)]
            blk = lax.shift_right_logical(vec, 7)
            m = (blk >= lo) & (blk < hi)
            cum = plsc.cumsum(m.astype(jnp.int32))
            pos = cnt + cum - 1
            plsc.store_scatter(jn, [pos], g * 16 + c0, mask=m)
            plsc.store_scatter(il, [pos], vec, mask=m)
            return cnt + _lane(cum, 15)

        def fire(bi, slot):
            blk_eff = jnp.minimum(lo + bi, n_full - 1)
            off = pl.multiple_of(blk_eff * 128, 128)
            pltpu.async_copy(
                ktT_hbm.at[:, pl.ds(off, 128)], bufk.at[slot], ksem.at[slot])
            pltpu.async_copy(
                vtT_hbm.at[:, pl.ds(off, 128)], bufv.at[slot], vsem.at[slot])

        for s in range(_RING):
            fire(s, s)

        cnt = lax.fori_loop(0, B // 16, collect, 0, unroll=4)

        def link(h, carry):
            nv = _lane(jn[pl.ds(h, 16)], 0)
            i = _lane(il[pl.ds(h, 16)], 0)
            b = lax.shift_right_logical(i, 7) - lo
            b_v = jnp.full((16,), b, jnp.int32)
            old = _lane(plsc.load_gather(head, [b_v]), 0)
            packed = nv | lax.shift_left(old, 14)
            plsc.store_scatter(jn, [jnp.full((16,), h, jnp.int32)],
                               jnp.full((16,), packed, jnp.int32), mask=lane0)
            plsc.store_scatter(head, [b_v],
                               jnp.full((16,), h, jnp.int32), mask=lane0)
            return carry

        lax.fori_loop(0, cnt, link, 0)

        def visit(h):
            nv = _lane(jn[pl.ds(h, 16)], 0)
            j = nv & (nullh - 1)
            nxt = lax.shift_right_logical(nv, 14)
            i = _lane(il[pl.ds(h, 16)], 0)
            return j, nxt, i

        def block_body(v, carry):
            for l in range(16):
                bi = v * 16 + l
                slot = l % _RING

                @pl.when(bi < nblk)
                def _():
                    pltpu.make_async_copy(
                        ktT_hbm.at[:, pl.ds(0, 128)], bufk.at[slot],
                        ksem.at[slot]).wait()
                    pltpu.make_async_copy(
                        vtT_hbm.at[:, pl.ds(0, 128)], bufv.at[slot],
                        vsem.at[slot]).wait()
                    bi_v = jnp.full((16,), bi, jnp.int32)
                    h0 = _lane(plsc.load_gather(head, [bi_v]), 0)
                    slot_v = jnp.full((16,), slot, jnp.int32)

                    def walk_body(carry):
                        h, steps = carry
                        j, nxt, i = visit(h)
                        lo_v = jnp.full((16,), i & 127, jnp.int32)
                        is_tail = jnp.full((16,), i >= tail_start, jnp.bool_)
                        toff = jnp.maximum(i - tail_start, 0) * D
                        hs = _lane(hs_v[pl.ds(0, 16)], 0)
                        so = hs & (_ORING - 1)

                        @pl.when((so == 0) & (hs >= _ORING))
                        def _():
                            for _s in range(_ORING):
                                pltpu.make_async_copy(
                                    stgk.at[pl.ds(0, D)],
                                    kout_hbm.at[pl.ds(0, D)], osemk).wait()
                                pltpu.make_async_copy(
                                    stgv.at[pl.ds(0, D)],
                                    vout_hbm.at[pl.ds(0, D)], osemv).wait()

                        for buf, tail_v, stg, out_hbm, osem in (
                            (bufk, tailk_v, stgk, kout_hbm, osemk),
                            (bufv, tailv_v, stgv, vout_hbm, osemv),
                        ):
                            v0 = plsc.load_gather(buf, [slot_v, c0, lo_v])
                            v1 = plsc.load_gather(buf, [slot_v, c1, lo_v])
                            t0 = plsc.load_gather(tail_v, [toff + c0])
                            t1 = plsc.load_gather(tail_v, [toff + c1])
                            stg[pl.ds(so * D, 16)] = jnp.where(is_tail, t0, v0)
                            stg[pl.ds(so * D + 16, 16)] = jnp.where(
                                is_tail, t1, v1)
                            pltpu.async_copy(
                                stg.at[pl.ds(so * D, D)],
                                out_hbm.at[pl.ds(j * D, D)], osem)
                        plsc.store_scatter(
                            hs_v, [zeros],
                            jnp.full((16,), hs + 1, jnp.int32), mask=lane0)
                        return nxt, steps + 1

                    lax.while_loop(
                        lambda c: (c[0] != nullh) & (c[1] < B),
                        walk_body, (h0, 0))

                    @pl.when(bi + _RING < nblk)
                    def _():
                        fire(bi + _RING, slot)

            return carry

        lax.fori_loop(0, 16, block_body, 0)

        hs_end = _lane(hs_v[pl.ds(0, 16)], 0)
        r_end = jnp.where(
            hs_end > 0, ((hs_end - 1) & (_ORING - 1)) + 1, 0)

        def final_drain(_, carry):
            pltpu.make_async_copy(
                stgk.at[pl.ds(0, D)], kout_hbm.at[pl.ds(0, D)], osemk).wait()
            pltpu.make_async_copy(
                stgv.at[pl.ds(0, D)], vout_hbm.at[pl.ds(0, D)], osemv).wait()
            return carry

        lax.fori_loop(0, r_end, final_drain, 0)

    return gather2


def kernel(x, key_table, value_table):
    B = x.shape[0]
    N, D = key_table.shape
    info = plsc.get_sparse_core_info()
    fn = _make_gather(B, D, N, info.num_cores, info.num_subcores)
    tail_start = N - (N % 128)
    xi = x.astype(jnp.int32).reshape(-1)
    ktail = key_table[tail_start:].reshape(-1)
    vtail = value_table[tail_start:].reshape(-1)
    kout, vout = fn(xi, key_table.T, value_table.T, ktail, vtail)
    return (kout.reshape(B, D), vout.reshape(B, D))
